# baseline (speedup 1.0000x reference)
import functools

import jax
import jax.numpy as jnp
from jax import lax
from jax.experimental import pallas as pl
from jax.experimental.pallas import tpu as pltpu
from jax.experimental.pallas import tpu_sc as plsc

NC = 2
NS = 16
NW = NC * NS
CHUNK = 128
NB = 3
DW = 16
NUM_GRAPHS = 64


def _fill_rows(ref, idx, width, value):
    vec = jnp.full((16,), value, jnp.float32)
    for j in range(width // 16):
        ref[idx, pl.ds(j * 16, 16)] = vec


def _sc_degree(n_pad, nch, g0, g1):
    rpt = n_pad // NS
    full, rem = rpt // CHUNK, rpt % CHUNK
    mesh = plsc.VectorSubcoreMesh(core_axis_name="c", subcore_axis_name="s")

    def body(dst3d, deg_out, didx, ones_v, zero_v, degsp, ssem):
        c = lax.axis_index("c")
        s = lax.axis_index("s")
        wid = c * NS + s

        def fill(i, _):
            _fill_rows(ones_v, i, DW, 1.0)
            _fill_rows(zero_v, i, DW, 0.0)
            return 0

        lax.fori_loop(0, CHUNK, fill, 0)
        pltpu.sync_copy(dst3d.at[wid], didx)
        base = s * rpt
        for k in range(full):
            pltpu.sync_copy(zero_v, degsp.at[pl.ds(base + k * CHUNK, CHUNK)])
        if rem:
            pltpu.sync_copy(zero_v.at[pl.ds(0, rem)],
                            degsp.at[pl.ds(base + full * CHUNK, rem)])
        plsc.subcore_barrier()

        def group(g, _):
            descs = []
            for b in range(NB):
                descs.append(pltpu.async_copy(
                    ones_v, degsp.at[didx.at[g * NB + b]], ssem, add=True))
            for d in descs:
                d.wait()
            return 0

        lax.fori_loop(0, jnp.where(c == 0, g0, g1), group, 0)
        plsc.subcore_barrier()
        pltpu.sync_copy(degsp.at[pl.ds(base, rpt)],
                        deg_out.at[c, pl.ds(base, rpt)])

    return pl.kernel(
        body,
        out_type=jax.ShapeDtypeStruct((NC, n_pad, DW), jnp.float32),
        mesh=mesh,
        scratch_types=[
            pltpu.VMEM((nch, CHUNK), jnp.int32),
            pltpu.VMEM((CHUNK, DW), jnp.float32),
            pltpu.VMEM((CHUNK, DW), jnp.float32),
            pltpu.VMEM_SHARED((n_pad, DW), jnp.float32),
            pltpu.SemaphoreType.DMA,
        ],
        compiler_params=pltpu.CompilerParams(use_tc_tiling_on_sc=False),
        name="sc_degree",
    )


def _sc_aggregate(n, n_pad, nch, feat, g0, g1, chunk=CHUNK, nb=NB):
    rpt = n_pad // NS
    full, rem = rpt // chunk, rpt % chunk
    mesh = plsc.VectorSubcoreMesh(core_axis_name="c", subcore_axis_name="s")

    def body(g_hbm, src3d, dst3d, acc_out, sidx, didx, rows, accsp,
             gsem, ssem):
        c = lax.axis_index("c")
        s = lax.axis_index("s")
        wid = c * NS + s
        sld = pltpu.async_copy(src3d.at[wid], sidx, gsem)
        dld = pltpu.async_copy(dst3d.at[wid], didx, ssem)

        def fill(i, _):
            _fill_rows(rows.at[0], i, feat, 0.0)
            return 0

        lax.fori_loop(0, chunk, fill, 0)
        base = s * rpt
        for k in range(full):
            pltpu.sync_copy(rows.at[0], accsp.at[pl.ds(base + k * chunk, chunk)])
        if rem:
            pltpu.sync_copy(rows.at[0, pl.ds(0, rem)],
                            accsp.at[pl.ds(base + full * chunk, rem)])
        sld.wait()
        dld.wait()
        plsc.subcore_barrier()

        def group(g, _):
            gd = []
            for b in range(nb):
                gd.append(pltpu.async_copy(
                    g_hbm.at[sidx.at[g * nb + b]], rows.at[b], gsem))
            sd = []
            for b in range(nb):
                gd[b].wait()
                sd.append(pltpu.async_copy(
                    rows.at[b], accsp.at[didx.at[g * nb + b]], ssem,
                    add=True))
            for d in sd:
                d.wait()
            return 0

        lax.fori_loop(0, jnp.where(c == 0, g0, g1), group, 0)
        plsc.subcore_barrier()
        pltpu.sync_copy(accsp.at[pl.ds(base, rpt)],
                        acc_out.at[c, pl.ds(base, rpt)])

    return pl.kernel(
        body,
        out_type=jax.ShapeDtypeStruct((NC, n_pad, feat), jnp.float32),
        mesh=mesh,
        scratch_types=[
            pltpu.VMEM((nch, chunk), jnp.int32),
            pltpu.VMEM((nch, chunk), jnp.int32),
            pltpu.VMEM((nb, chunk, feat), jnp.float32),
            pltpu.VMEM_SHARED((n_pad, feat), jnp.float32),
            pltpu.SemaphoreType.DMA,
            pltpu.SemaphoreType.DMA,
        ],
        compiler_params=pltpu.CompilerParams(use_tc_tiling_on_sc=False),
        name=f"sc_aggregate_{feat}",
    )


def _tc_pre(n, n_pad, f_in, f_out):
    def body(deg_ref, x_ref, w_ref, g_ref, dis_ref):
        deg = deg_ref[0, :n, 0:1] + deg_ref[1, :n, 0:1]
        dis = lax.rsqrt(jnp.maximum(deg, 1.0))
        dis_ref[...] = dis
        g_ref[...] = jnp.dot(dis * x_ref[...], w_ref[...],
                             precision=lax.Precision.HIGHEST)

    return pl.pallas_call(
        body,
        out_shape=(jax.ShapeDtypeStruct((n, f_out), jnp.float32),
                   jax.ShapeDtypeStruct((n, 1), jnp.float32)),
    )


def _tc_mid(n, n_pad, f_in, f_out):
    def body(acc_ref, dis_ref, b_ref, w_ref, out_ref):
        a = acc_ref[0, :n, :] + acc_ref[1, :n, :]
        dis = dis_ref[...]
        y = jnp.maximum(dis * a + b_ref[...], 0.0)
        out_ref[...] = jnp.dot(dis * y, w_ref[...],
                               precision=lax.Precision.HIGHEST)

    return pl.pallas_call(
        body,
        out_shape=jax.ShapeDtypeStruct((n, f_out), jnp.float32),
    )


def _tc_mid2(n, n_pad, f_half, f_out):
    def body(acc_a, acc_b, dis_ref, b_ref, w_ref, out_ref):
        a = jnp.concatenate(
            [acc_a[0, :n, :] + acc_a[1, :n, :],
             acc_b[0, :n, :] + acc_b[1, :n, :]], axis=1)
        dis = dis_ref[...]
        y = jnp.maximum(dis * a + b_ref[...], 0.0)
        out_ref[...] = jnp.dot(dis * y, w_ref[...],
                               precision=lax.Precision.HIGHEST)

    return pl.pallas_call(
        body,
        out_shape=jax.ShapeDtypeStruct((n, f_out), jnp.float32),
    )


def _tc_final(n, n_pad, feat, ncls):
    def body(acc_ref, dis_ref, b_ref, batch_ref, wfc_ref, bfc_ref, out_ref):
        a = acc_ref[0, :n, :] + acc_ref[1, :n, :]
        y = jnp.maximum(dis_ref[...] * a + b_ref[...], 0.0)
        gids = batch_ref[...]
        onehot = (gids == lax.broadcasted_iota(
            jnp.int32, (n, NUM_GRAPHS), 1)).astype(jnp.float32)
        sums = lax.dot_general(onehot, y, (((0,), (0,)), ((), ())),
                               precision=lax.Precision.HIGHEST)
        counts = lax.dot_general(onehot, jnp.ones((n, 1), jnp.float32),
                                 (((0,), (0,)), ((), ())),
                                 precision=lax.Precision.HIGHEST)
        pooled = sums / jnp.maximum(counts, 1.0)
        logits = jnp.dot(pooled, wfc_ref[...],
                         precision=lax.Precision.HIGHEST) + bfc_ref[...]
        m = jnp.max(logits, axis=1, keepdims=True)
        lse = jnp.log(jnp.sum(jnp.exp(logits - m), axis=1, keepdims=True)) + m
        out_ref[...] = logits - lse

    return pl.pallas_call(
        body,
        out_shape=jax.ShapeDtypeStruct((NUM_GRAPHS, ncls), jnp.float32),
    )


def kernel(x, edge_index, batch, W1, b1, W2, b2, W3, b3, Wfc, bfc):
    n, f_in = x.shape
    e = edge_index.shape[1]

    loop = jnp.arange(n, dtype=jnp.int32)
    src = jnp.concatenate([edge_index[0], loop])
    dst = jnp.concatenate([edge_index[1], loop])
    etot = e + n
    nb64 = 3
    G0, G1 = 21, 33
    nch = max(G0, G1) * nb64
    e0 = G0 * nb64 * CHUNK
    e1 = G1 * nb64 * CHUNK
    pad = NS * (e0 + e1) - etot
    n_pad = -(-(n + 1) // (NS * 8)) * (NS * 8)
    pad_dst = n + jnp.arange(pad, dtype=jnp.int32) % (n_pad - n)
    src_p = jnp.concatenate([src, jnp.zeros((pad,), jnp.int32)])
    dst_p = jnp.concatenate([dst, pad_dst])

    def split3d(a, fill):
        a0 = a[:NS * e0].reshape(NS, G0 * nb64, CHUNK)
        a1 = a[NS * e0:].reshape(NS, G1 * nb64, CHUNK)
        f0 = jnp.full((NS, nch - G0 * nb64, CHUNK), fill, jnp.int32)
        f1 = jnp.full((NS, nch - G1 * nb64, CHUNK), fill, jnp.int32)
        return jnp.concatenate([jnp.concatenate([a0, f0], axis=1),
                                jnp.concatenate([a1, f1], axis=1)], axis=0)

    src3d = split3d(src_p, 0)
    dst3d = split3d(dst_p, n)
    nch64 = 2 * nch
    src3d_64 = src3d.reshape(NW, nch64, 64)
    dst3d_64 = dst3d.reshape(NW, nch64, 64)

    deg2 = _sc_degree(n_pad, nch, G0, G1)(dst3d)
    g1, dis = _tc_pre(n, n_pad, f_in, W1.shape[1])(deg2, x, W1)
    a1 = _sc_aggregate(n, n_pad, nch, W1.shape[1], G0, G1, nb=nb64)(
        g1, src3d, dst3d)
    g2 = _tc_mid(n, n_pad, W1.shape[1], W2.shape[1])(a1, dis, b1, W2)
    a2 = _sc_aggregate(n, n_pad, nch64, W2.shape[1], 2 * G0, 2 * G1,
                       chunk=64)(g2, src3d_64, dst3d_64)
    g3 = _tc_mid(n, n_pad, W2.shape[1], W3.shape[1])(a2, dis, b2, W3)
    a3 = _sc_aggregate(n, n_pad, nch, W3.shape[1], G0, G1, nb=nb64)(
        g3, src3d, dst3d)
    out = _tc_final(n, n_pad, W3.shape[1], Wfc.shape[1])(
        a3, dis, b3, batch.reshape(n, 1).astype(jnp.int32), Wfc, bfc)
    return out


# speedup vs baseline: 1.1164x; 1.1164x over previous
import functools

import jax
import jax.numpy as jnp
from jax import lax
from jax.experimental import pallas as pl
from jax.experimental.pallas import tpu as pltpu
from jax.experimental.pallas import tpu_sc as plsc

NC = 2
NS = 16
NW = NC * NS
CHUNK = 128
NB = 3
DW = 16
NUM_GRAPHS = 64


def _fill_rows(ref, idx, width, value):
    vec = jnp.full((16,), value, jnp.float32)
    for j in range(width // 16):
        ref[idx, pl.ds(j * 16, 16)] = vec


def _sc_degree(n_pad, nch, g0, g1):
    rpt = n_pad // NS
    full, rem = rpt // CHUNK, rpt % CHUNK
    mesh = plsc.VectorSubcoreMesh(core_axis_name="c", subcore_axis_name="s")

    def body(dst3d, deg_out, didx, ones_v, zero_v, degsp, ssem):
        c = lax.axis_index("c")
        s = lax.axis_index("s")
        wid = c * NS + s

        def fill(i, _):
            _fill_rows(ones_v, i, DW, 1.0)
            _fill_rows(zero_v, i, DW, 0.0)
            return 0

        lax.fori_loop(0, CHUNK, fill, 0)
        pltpu.sync_copy(dst3d.at[wid], didx)
        base = s * rpt
        for k in range(full):
            pltpu.sync_copy(zero_v, degsp.at[pl.ds(base + k * CHUNK, CHUNK)])
        if rem:
            pltpu.sync_copy(zero_v.at[pl.ds(0, rem)],
                            degsp.at[pl.ds(base + full * CHUNK, rem)])
        plsc.subcore_barrier()

        def group(g, _):
            descs = []
            for b in range(NB):
                descs.append(pltpu.async_copy(
                    ones_v, degsp.at[didx.at[g * NB + b]], ssem, add=True))
            for d in descs:
                d.wait()
            return 0

        lax.fori_loop(0, jnp.where(c == 0, g0, g1), group, 0)
        plsc.subcore_barrier()
        pltpu.sync_copy(degsp.at[pl.ds(base, rpt)],
                        deg_out.at[c, pl.ds(base, rpt)])

    return pl.kernel(
        body,
        out_type=jax.ShapeDtypeStruct((NC, n_pad, DW), jnp.float32),
        mesh=mesh,
        scratch_types=[
            pltpu.VMEM((nch, CHUNK), jnp.int32),
            pltpu.VMEM((CHUNK, DW), jnp.float32),
            pltpu.VMEM((CHUNK, DW), jnp.float32),
            pltpu.VMEM_SHARED((n_pad, DW), jnp.float32),
            pltpu.SemaphoreType.DMA,
        ],
        compiler_params=pltpu.CompilerParams(use_tc_tiling_on_sc=False),
        name="sc_degree",
    )


def _sc_aggregate(n, n_pad, nch, feat, g0, g1, chunk=CHUNK, nb=NB):
    rpt = n_pad // NS
    full, rem = rpt // chunk, rpt % chunk
    mesh = plsc.VectorSubcoreMesh(core_axis_name="c", subcore_axis_name="s")

    def body(g_hbm, src3d, dst3d, acc_out, sidx, didx, rows, accsp,
             gsem, ssem):
        c = lax.axis_index("c")
        s = lax.axis_index("s")
        wid = c * NS + s
        sld = pltpu.async_copy(src3d.at[wid], sidx, gsem)
        dld = pltpu.async_copy(dst3d.at[wid], didx, ssem)

        def fill(i, _):
            _fill_rows(rows.at[0], i, feat, 0.0)
            return 0

        lax.fori_loop(0, chunk, fill, 0)
        base = s * rpt
        for k in range(full):
            pltpu.sync_copy(rows.at[0], accsp.at[pl.ds(base + k * chunk, chunk)])
        if rem:
            pltpu.sync_copy(rows.at[0, pl.ds(0, rem)],
                            accsp.at[pl.ds(base + full * chunk, rem)])
        sld.wait()
        dld.wait()
        plsc.subcore_barrier()

        def group(g, _):
            gd = []
            for b in range(nb):
                gd.append(pltpu.async_copy(
                    g_hbm.at[sidx.at[g * nb + b]], rows.at[b], gsem))
            sd = []
            for b in range(nb):
                gd[b].wait()
                sd.append(pltpu.async_copy(
                    rows.at[b], accsp.at[didx.at[g * nb + b]], ssem,
                    add=True))
            for d in sd:
                d.wait()
            return 0

        lax.fori_loop(0, jnp.where(c == 0, g0, g1), group, 0)
        plsc.subcore_barrier()
        pltpu.sync_copy(accsp.at[pl.ds(base, rpt)],
                        acc_out.at[c, pl.ds(base, rpt)])

    return pl.kernel(
        body,
        out_type=jax.ShapeDtypeStruct((NC, n_pad, feat), jnp.float32),
        mesh=mesh,
        scratch_types=[
            pltpu.VMEM((nch, chunk), jnp.int32),
            pltpu.VMEM((nch, chunk), jnp.int32),
            pltpu.VMEM((nb, chunk, feat), jnp.float32),
            pltpu.VMEM_SHARED((n_pad, feat), jnp.float32),
            pltpu.SemaphoreType.DMA,
            pltpu.SemaphoreType.DMA,
        ],
        compiler_params=pltpu.CompilerParams(use_tc_tiling_on_sc=False),
        name=f"sc_aggregate_{feat}",
    )


def _tc_pre(n, n_pad, f_in, f_out):
    def body(deg_ref, x_ref, w_ref, g_ref, dis_ref):
        deg = deg_ref[0, :n, 0:1] + deg_ref[1, :n, 0:1]
        dis = lax.rsqrt(jnp.maximum(deg, 1.0))
        dis_ref[...] = dis
        g_ref[...] = jnp.dot(dis * x_ref[...], w_ref[...],
                             precision=lax.Precision.HIGHEST)

    return pl.pallas_call(
        body,
        out_shape=(jax.ShapeDtypeStruct((n, f_out), jnp.float32),
                   jax.ShapeDtypeStruct((n, 1), jnp.float32)),
    )


def _tc_mid(n, n_pad, f_in, f_out):
    def body(acc_ref, dis_ref, b_ref, w_ref, out_ref):
        a = acc_ref[0, :n, :] + acc_ref[1, :n, :]
        dis = dis_ref[...]
        y = jnp.maximum(dis * a + b_ref[...], 0.0)
        out_ref[...] = jnp.dot(dis * y, w_ref[...],
                               precision=lax.Precision.HIGHEST)

    return pl.pallas_call(
        body,
        out_shape=jax.ShapeDtypeStruct((n, f_out), jnp.float32),
    )


def _tc_mid2(n, n_pad, f_half, f_out):
    def body(acc_a, acc_b, dis_ref, b_ref, w_ref, out_ref):
        a = jnp.concatenate(
            [acc_a[0, :n, :] + acc_a[1, :n, :],
             acc_b[0, :n, :] + acc_b[1, :n, :]], axis=1)
        dis = dis_ref[...]
        y = jnp.maximum(dis * a + b_ref[...], 0.0)
        out_ref[...] = jnp.dot(dis * y, w_ref[...],
                               precision=lax.Precision.HIGHEST)

    return pl.pallas_call(
        body,
        out_shape=jax.ShapeDtypeStruct((n, f_out), jnp.float32),
    )


def _tc_final(n, n_pad, feat, ncls):
    def body(acc_ref, dis_ref, b_ref, batch_ref, wfc_ref, bfc_ref, out_ref):
        a = acc_ref[0, :n, :] + acc_ref[1, :n, :]
        y = jnp.maximum(dis_ref[...] * a + b_ref[...], 0.0)
        gids = batch_ref[...]
        onehot = (gids == lax.broadcasted_iota(
            jnp.int32, (n, NUM_GRAPHS), 1)).astype(jnp.float32)
        sums = lax.dot_general(onehot, y, (((0,), (0,)), ((), ())),
                               precision=lax.Precision.HIGHEST)
        counts = lax.dot_general(onehot, jnp.ones((n, 1), jnp.float32),
                                 (((0,), (0,)), ((), ())),
                                 precision=lax.Precision.HIGHEST)
        pooled = sums / jnp.maximum(counts, 1.0)
        logits = jnp.dot(pooled, wfc_ref[...],
                         precision=lax.Precision.HIGHEST) + bfc_ref[...]
        m = jnp.max(logits, axis=1, keepdims=True)
        lse = jnp.log(jnp.sum(jnp.exp(logits - m), axis=1, keepdims=True)) + m
        out_ref[...] = logits - lse

    return pl.pallas_call(
        body,
        out_shape=jax.ShapeDtypeStruct((NUM_GRAPHS, ncls), jnp.float32),
    )


def kernel(x, edge_index, batch, W1, b1, W2, b2, W3, b3, Wfc, bfc):
    n, f_in = x.shape
    e = edge_index.shape[1]

    loop = jnp.arange(n, dtype=jnp.int32)
    src = jnp.concatenate([edge_index[0], loop])
    dst = jnp.concatenate([edge_index[1], loop])
    etot = e + n
    nb64 = 3
    G0, G1 = 33, 21
    nch = max(G0, G1) * nb64
    e0 = G0 * nb64 * CHUNK
    e1 = G1 * nb64 * CHUNK
    pad = NS * (e0 + e1) - etot
    n_pad = -(-(n + 1) // (NS * 8)) * (NS * 8)
    pad_dst = n + jnp.arange(pad, dtype=jnp.int32) % (n_pad - n)
    src_p = jnp.concatenate([src, jnp.zeros((pad,), jnp.int32)])
    dst_p = jnp.concatenate([dst, pad_dst])

    def split3d(a, fill):
        a0 = a[:NS * e0].reshape(NS, G0 * nb64, CHUNK)
        a1 = a[NS * e0:].reshape(NS, G1 * nb64, CHUNK)
        f0 = jnp.full((NS, nch - G0 * nb64, CHUNK), fill, jnp.int32)
        f1 = jnp.full((NS, nch - G1 * nb64, CHUNK), fill, jnp.int32)
        return jnp.concatenate([jnp.concatenate([a0, f0], axis=1),
                                jnp.concatenate([a1, f1], axis=1)], axis=0)

    src3d = split3d(src_p, 0)
    dst3d = split3d(dst_p, n)
    nch64 = 2 * nch
    src3d_64 = src3d.reshape(NW, nch64, 64)
    dst3d_64 = dst3d.reshape(NW, nch64, 64)

    deg2 = _sc_degree(n_pad, nch, G0, G1)(dst3d)
    g1, dis = _tc_pre(n, n_pad, f_in, W1.shape[1])(deg2, x, W1)
    a1 = _sc_aggregate(n, n_pad, nch, W1.shape[1], G0, G1, nb=nb64)(
        g1, src3d, dst3d)
    g2 = _tc_mid(n, n_pad, W1.shape[1], W2.shape[1])(a1, dis, b1, W2)
    a2 = _sc_aggregate(n, n_pad, nch64, W2.shape[1], 2 * G0, 2 * G1,
                       chunk=64)(g2, src3d_64, dst3d_64)
    g3 = _tc_mid(n, n_pad, W2.shape[1], W3.shape[1])(a2, dis, b2, W3)
    a3 = _sc_aggregate(n, n_pad, nch, W3.shape[1], G0, G1, nb=nb64)(
        g3, src3d, dst3d)
    out = _tc_final(n, n_pad, W3.shape[1], Wfc.shape[1])(
        a3, dis, b3, batch.reshape(n, 1).astype(jnp.int32), Wfc, bfc)
    return out
